# baseline (speedup 1.0000x reference)
import functools

import jax
import jax.numpy as jnp
from jax.experimental import pallas as pl
from jax.experimental.pallas import tpu as pltpu

E = 8
TOPK = 2
D = 1024
FF = 2048
T = 2048

BT = 256
F = 512
NFF = FF // F
NT = T // BT


def _routing_weights(x_tile, router_w):
    logits = jax.lax.dot_general(
        x_tile, router_w, (((1,), (0,)), ((), ())),
        preferred_element_type=jnp.float32)
    m1 = jnp.max(logits, axis=1, keepdims=True)
    e_iota = jax.lax.broadcasted_iota(jnp.int32, logits.shape, 1)
    big = jnp.int32(E)
    i1 = jnp.min(jnp.where(logits == m1, e_iota, big), axis=1, keepdims=True)
    masked = jnp.where(e_iota == i1, -jnp.inf, logits)
    m2 = jnp.max(masked, axis=1, keepdims=True)
    i2 = jnp.min(jnp.where(masked == m2, e_iota, big), axis=1, keepdims=True)
    w1 = 1.0 / (1.0 + jnp.exp(m2 - m1))
    w2 = 1.0 - w1
    w_all = jnp.where(e_iota == i1, w1, 0.0) + jnp.where(e_iota == i2, w2, 0.0)
    return w_all.astype(jnp.float32)


def _moe_body(x_ref, rw_ref, wg_ref, wu_ref, wd_ref, out_ref,
              acc_ref, w_ref):
    e = pl.program_id(0)
    j = pl.program_id(1)
    i = pl.program_id(2)

    @pl.when(jnp.logical_and(e == 0, j == 0))
    def _():
        w_ref[i] = _routing_weights(x_ref[...], rw_ref[...])
        acc_ref[i] = jnp.zeros_like(acc_ref[i])

    x = x_ref[...]
    g = jax.lax.dot_general(x, wg_ref[0], (((1,), (0,)), ((), ())),
                            preferred_element_type=jnp.float32)
    u = jax.lax.dot_general(x, wu_ref[0], (((1,), (0,)), ((), ())),
                            preferred_element_type=jnp.float32)
    gated = (g * jax.lax.logistic(g)) * u
    w_tile = w_ref[i]
    e_onehot = (jax.lax.broadcasted_iota(jnp.int32, w_tile.shape, 1) == e)
    w_e = jnp.sum(jnp.where(e_onehot, w_tile, 0.0), axis=1, keepdims=True)
    contrib = jax.lax.dot_general(gated * w_e, wd_ref[0],
                                  (((1,), (0,)), ((), ())),
                                  preferred_element_type=jnp.float32)
    acc_ref[i] += contrib

    @pl.when(jnp.logical_and(e == E - 1, j == NFF - 1))
    def _():
        out_ref[...] = acc_ref[i]


@jax.jit
def kernel(hidden_states, router_w, gate_up_proj, down_proj):
    b, s, d = hidden_states.shape
    flat = hidden_states.reshape(-1, d)

    grid = (E, NFF, NT)
    out = pl.pallas_call(
        _moe_body,
        grid=grid,
        in_specs=[
            pl.BlockSpec((BT, D), lambda e, j, i: (i, 0)),
            pl.BlockSpec((D, E), lambda e, j, i: (0, 0)),
            pl.BlockSpec((1, D, F), lambda e, j, i: (e, 0, j)),
            pl.BlockSpec((1, D, F), lambda e, j, i: (e, 0, j + NFF)),
            pl.BlockSpec((1, F, D), lambda e, j, i: (e, j, 0)),
        ],
        out_specs=pl.BlockSpec((BT, D), lambda e, j, i: (i, 0)),
        out_shape=jax.ShapeDtypeStruct((T, D), jnp.float32),
        scratch_shapes=[
            pltpu.VMEM((NT, BT, D), jnp.float32),
            pltpu.VMEM((NT, BT, E), jnp.float32),
        ],
        compiler_params=pltpu.CompilerParams(
            dimension_semantics=("arbitrary", "arbitrary", "arbitrary"),
        ),
    )(flat, router_w, gate_up_proj, gate_up_proj, down_proj)
    return out.reshape(b, s, d)


# speedup vs baseline: 1.9565x; 1.9565x over previous
import functools

import jax
import jax.numpy as jnp
from jax import lax
from jax.experimental import pallas as pl
from jax.experimental.pallas import tpu as pltpu
from jax.experimental.pallas import tpu_sc as plsc

E = 8
D = 1024
FF = 2048
T = 2048
A = 2 * T
B = 512
G = A // B + E
P = G * B
F = 512
NFF = FF // F

NW = 32
TPW = T // NW


def _router_body(x_ref, rw_ref, pos_ref, w2_ref, te_ref, xg_ref, valid_ref):
    x = x_ref[...]
    logits = lax.dot_general(x, rw_ref[...], (((1,), (0,)), ((), ())),
                             preferred_element_type=jnp.float32)
    e_iota = lax.broadcasted_iota(jnp.int32, (T, E), 1)
    m1 = jnp.max(logits, axis=1, keepdims=True)
    i1 = jnp.min(jnp.where(logits == m1, e_iota, E), axis=1, keepdims=True)
    masked = jnp.where(e_iota == i1, -jnp.inf, logits)
    m2 = jnp.max(masked, axis=1, keepdims=True)
    i2 = jnp.min(jnp.where(masked == m2, e_iota, E), axis=1, keepdims=True)
    w1 = 1.0 / (1.0 + jnp.exp(m2 - m1))
    w2 = 1.0 - w1

    sel1 = (e_iota == i1).astype(jnp.float32)
    sel2 = (e_iota == i2).astype(jnp.float32)
    s = sel1 + sel2

    t_row = lax.broadcasted_iota(jnp.int32, (T, T), 0)
    t_col = lax.broadcasted_iota(jnp.int32, (T, T), 1)
    lt = (t_col < t_row).astype(jnp.float32)
    csum = lax.dot_general(lt, s, (((1,), (0,)), ((), ())),
                           preferred_element_type=jnp.float32)
    counts = csum[T - 1:T, :] + s[T - 1:T, :]

    cnt_pad = jnp.ceil(counts / B) * B
    e_row = lax.broadcasted_iota(jnp.int32, (E, E), 0)
    e_col = lax.broadcasted_iota(jnp.int32, (E, E), 1)
    lt_e = (e_row < e_col).astype(jnp.float32)
    base = lax.dot_general(cnt_pad, lt_e, (((1,), (0,)), ((), ())),
                           preferred_element_type=jnp.float32)
    total_pad = base[0, E - 1] + cnt_pad[0, E - 1]

    rank1 = jnp.sum(csum * sel1, axis=1, keepdims=True)
    rank2 = jnp.sum(csum * sel2, axis=1, keepdims=True)
    base1 = jnp.sum(base * sel1, axis=1, keepdims=True)
    base2 = jnp.sum(base * sel2, axis=1, keepdims=True)
    pos1 = (base1 + rank1)[:, 0]
    pos2_ = (base2 + rank2)[:, 0]
    pos_ref[0, :] = pos1.astype(jnp.int32)
    pos_ref[1, :] = pos2_.astype(jnp.int32)
    w2_ref[0, :] = w1[:, 0]
    w2_ref[1, :] = w2[:, 0]

    nvalid = total_pad / B
    g_iota = lax.broadcasted_iota(jnp.int32, (G, 1), 0).astype(jnp.float32)
    validg = g_iota < nvalid
    gs = jnp.minimum(g_iota * B, total_pad - B)
    ge = jnp.sum((jnp.broadcast_to(base, (G, E)) <= gs).astype(jnp.int32),
                 axis=1, keepdims=True) - 1
    xg = jnp.minimum(g_iota, nvalid - 1.0)
    te_ref[...] = jnp.reshape(ge, (1, G)).astype(jnp.int32)
    xg_ref[...] = jnp.reshape(xg, (1, G)).astype(jnp.int32)
    valid_ref[...] = jnp.reshape(validg, (1, G)).astype(jnp.int32)


def _router(flat, router_w):
    return pl.pallas_call(
        _router_body,
        in_specs=[
            pl.BlockSpec((T, D), lambda: (0, 0)),
            pl.BlockSpec((D, E), lambda: (0, 0)),
        ],
        out_specs=[
            pl.BlockSpec((2, T), lambda: (0, 0)),
            pl.BlockSpec((2, T), lambda: (0, 0)),
            pl.BlockSpec((1, G), lambda: (0, 0)),
            pl.BlockSpec((1, G), lambda: (0, 0)),
            pl.BlockSpec((1, G), lambda: (0, 0)),
        ],
        out_shape=[
            jax.ShapeDtypeStruct((2, T), jnp.int32),
            jax.ShapeDtypeStruct((2, T), jnp.float32),
            jax.ShapeDtypeStruct((1, G), jnp.int32),
            jax.ShapeDtypeStruct((1, G), jnp.int32),
            jax.ShapeDtypeStruct((1, G), jnp.int32),
        ],
    )(flat, router_w)


def _dispatch_sc(flat, pos, w2):
    mesh = plsc.VectorSubcoreMesh(core_axis_name="c", subcore_axis_name="s")

    @functools.partial(
        pl.kernel, mesh=mesh,
        out_type=(
            jax.ShapeDtypeStruct((P, D), jnp.float32),
            jax.ShapeDtypeStruct((P,), jnp.float32),
        ),
        scratch_types=[
            pltpu.VMEM((TPW,), jnp.int32),
            pltpu.VMEM((TPW,), jnp.int32),
            pltpu.VMEM((TPW,), jnp.float32),
            pltpu.VMEM((TPW,), jnp.float32),
            pltpu.VMEM((TPW, D), jnp.float32),
            pltpu.SemaphoreType.DMA,
        ],
    )
    def k(flat_hbm, pos_hbm, w2_hbm, xs_hbm, ws_hbm,
          idx0_v, idx1_v, wv0_v, wv1_v, rows_v, sem):
        wid = lax.axis_index("s") * 2 + lax.axis_index("c")
        tbase = wid * TPW
        pltpu.sync_copy(pos_hbm.at[0, pl.ds(tbase, TPW)], idx0_v)
        pltpu.sync_copy(pos_hbm.at[1, pl.ds(tbase, TPW)], idx1_v)
        pltpu.sync_copy(w2_hbm.at[0, pl.ds(tbase, TPW)], wv0_v)
        pltpu.sync_copy(w2_hbm.at[1, pl.ds(tbase, TPW)], wv1_v)
        pltpu.sync_copy(flat_hbm.at[pl.ds(tbase, TPW)], rows_v)
        c0 = pltpu.async_copy(rows_v, xs_hbm.at[idx0_v], sem)
        c1 = pltpu.async_copy(rows_v, xs_hbm.at[idx1_v], sem)
        c2 = pltpu.async_copy(wv0_v, ws_hbm.at[idx0_v], sem)
        c3 = pltpu.async_copy(wv1_v, ws_hbm.at[idx1_v], sem)
        c0.wait()
        c1.wait()
        c2.wait()
        c3.wait()

    return k(flat, pos, w2)


def _gemm_body(te_ref, xg_ref, valid_ref, xs_ref, wg_ref, wu_ref, wd_ref,
               ws_ref, y_ref, acc_ref):
    g = pl.program_id(0)
    j = pl.program_id(1)

    @pl.when(valid_ref[g] == 1)
    def _():
        x = xs_ref[...]
        gate = lax.dot_general(x, wg_ref[0], (((1,), (0,)), ((), ())),
                               preferred_element_type=jnp.float32)
        up = lax.dot_general(x, wu_ref[0], (((1,), (0,)), ((), ())),
                             preferred_element_type=jnp.float32)
        gated = (gate * lax.logistic(gate)) * up
        contrib = lax.dot_general(gated, wd_ref[0], (((1,), (0,)), ((), ())),
                                  preferred_element_type=jnp.float32)

        @pl.when(j == 0)
        def _():
            acc_ref[...] = jnp.zeros_like(acc_ref)

        acc_ref[...] += contrib

        @pl.when(j == NFF - 1)
        def _():
            y_ref[...] = acc_ref[...] * ws_ref[...]


def _gemm(te, xg, valid, xs, gate_up_proj, down_proj, ws):
    def jc(j, valid_g):
        return jnp.where(valid_g == 1, j, NFF - 1)

    grid_spec = pltpu.PrefetchScalarGridSpec(
        num_scalar_prefetch=3,
        grid=(G, NFF),
        in_specs=[
            pl.BlockSpec((B, D), lambda g, j, te, xg, v: (xg[g], 0)),
            pl.BlockSpec((1, D, F),
                         lambda g, j, te, xg, v: (te[g], 0, jc(j, v[g]))),
            pl.BlockSpec((1, D, F),
                         lambda g, j, te, xg, v: (te[g], 0, jc(j, v[g]) + NFF)),
            pl.BlockSpec((1, F, D),
                         lambda g, j, te, xg, v: (te[g], jc(j, v[g]), 0)),
            pl.BlockSpec((B, 1), lambda g, j, te, xg, v: (xg[g], 0)),
        ],
        out_specs=pl.BlockSpec((B, D), lambda g, j, te, xg, v: (xg[g], 0)),
        scratch_shapes=[pltpu.VMEM((B, D), jnp.float32)],
    )
    return pl.pallas_call(
        _gemm_body,
        grid_spec=grid_spec,
        out_shape=jax.ShapeDtypeStruct((P, D), jnp.float32),
        compiler_params=pltpu.CompilerParams(
            dimension_semantics=("arbitrary", "arbitrary"),
        ),
    )(te, xg, valid, xs, gate_up_proj, gate_up_proj, down_proj,
      ws.reshape(P, 1))


_CH = 32


def _combine_sc(y, pos):
    mesh = plsc.VectorSubcoreMesh(core_axis_name="c", subcore_axis_name="s")

    @functools.partial(
        pl.kernel, mesh=mesh,
        out_type=jax.ShapeDtypeStruct((T, D), jnp.float32),
        scratch_types=[
            pltpu.VMEM((_CH,), jnp.int32),
            pltpu.VMEM((_CH,), jnp.int32),
            pltpu.VMEM((_CH, D), jnp.float32),
            pltpu.VMEM((_CH, D), jnp.float32),
            pltpu.SemaphoreType.DMA,
        ],
    )
    def k(y_hbm, pos_hbm, out_hbm, idx0_v, idx1_v, y0_v, y1_v, sem):
        wid = lax.axis_index("s") * 2 + lax.axis_index("c")
        for ch in range(TPW // _CH):
            tbase = wid * TPW + ch * _CH
            pltpu.sync_copy(pos_hbm.at[0, pl.ds(tbase, _CH)], idx0_v)
            pltpu.sync_copy(pos_hbm.at[1, pl.ds(tbase, _CH)], idx1_v)
            c0 = pltpu.async_copy(y_hbm.at[idx0_v], y0_v, sem)
            c1 = pltpu.async_copy(y_hbm.at[idx1_v], y1_v, sem)
            c0.wait()
            c1.wait()

            def row(r, _):
                for c in range(D // 16):
                    y0_v[r, pl.ds(c * 16, 16)] += y1_v[r, pl.ds(c * 16, 16)]
                return 0

            lax.fori_loop(0, _CH, row, 0)
            pltpu.sync_copy(y0_v, out_hbm.at[pl.ds(tbase, _CH)])

    return k(y, pos)


@jax.jit
def kernel(hidden_states, router_w, gate_up_proj, down_proj):
    b, s, d = hidden_states.shape
    flat = hidden_states.reshape(-1, d)
    pos, w2, te, xg, valid = _router(flat, router_w)
    xs, ws = _dispatch_sc(flat, pos, w2)
    y = _gemm(te.reshape(G), xg.reshape(G), valid.reshape(G),
              xs, gate_up_proj, down_proj, ws)
    out = _combine_sc(y, pos)
    return out.reshape(b, s, d)


# speedup vs baseline: 2.0259x; 1.0355x over previous
import functools

import numpy as _np

import jax
import jax.numpy as jnp
from jax import lax
from jax.experimental import pallas as pl
from jax.experimental.pallas import tpu as pltpu
from jax.experimental.pallas import tpu_sc as plsc

E = 8
D = 1024
FF = 2048
T = 2048
A = 2 * T
B = 512
G = A // B + E
P = G * B
F = 512
NFF = FF // F

NW = 32
TPW = T // NW


def _router_body(x_ref, rw_ref, ut_ref, pos_ref, w2_ref, te_ref, xg_ref,
                 valid_ref):
    lg = lax.dot_general(rw_ref[...], x_ref[...], (((0,), (1,)), ((), ())),
                         preferred_element_type=jnp.float32)
    e_iota = lax.broadcasted_iota(jnp.int32, (E, T), 0)
    m1 = jnp.max(lg, axis=0, keepdims=True)
    i1 = jnp.min(jnp.where(lg == m1, e_iota, E), axis=0, keepdims=True)
    masked = jnp.where(e_iota == i1, -jnp.inf, lg)
    m2 = jnp.max(masked, axis=0, keepdims=True)
    i2 = jnp.min(jnp.where(masked == m2, e_iota, E), axis=0, keepdims=True)
    w1 = 1.0 / (1.0 + jnp.exp(m2 - m1))
    w2 = 1.0 - w1

    sel1 = (e_iota == i1).astype(jnp.float32)
    sel2 = (e_iota == i2).astype(jnp.float32)
    s = sel1 + sel2

    csum = lax.dot_general(s.astype(jnp.bfloat16), ut_ref[...],
                           (((1,), (0,)), ((), ())),
                           preferred_element_type=jnp.float32)
    counts = csum[:, T - 1:T] + s[:, T - 1:T]

    cnt_pad = jnp.ceil(counts / B) * B
    e_row = lax.broadcasted_iota(jnp.int32, (E, E), 0)
    e_col = lax.broadcasted_iota(jnp.int32, (E, E), 1)
    lt_e = (e_col < e_row).astype(jnp.float32)
    base = lax.dot_general(lt_e, cnt_pad, (((1,), (0,)), ((), ())),
                           preferred_element_type=jnp.float32)
    total_pad = base[E - 1, 0] + cnt_pad[E - 1, 0]

    rank1 = jnp.sum(csum * sel1, axis=0, keepdims=True)
    rank2 = jnp.sum(csum * sel2, axis=0, keepdims=True)
    base1 = jnp.sum(base * sel1, axis=0, keepdims=True)
    base2 = jnp.sum(base * sel2, axis=0, keepdims=True)
    pos_ref[0:1, :] = (base1 + rank1).astype(jnp.int32)
    pos_ref[1:2, :] = (base2 + rank2).astype(jnp.int32)
    w2_ref[0:1, :] = w1
    w2_ref[1:2, :] = w2

    nvalid = total_pad / B
    g_iota = lax.broadcasted_iota(jnp.int32, (1, G), 1).astype(jnp.float32)
    validg = g_iota < nvalid
    gs = jnp.minimum(g_iota * B, total_pad - B)
    ge = jnp.sum((jnp.broadcast_to(base, (E, G)) <= gs).astype(jnp.int32),
                 axis=0, keepdims=True) - 1
    xg = jnp.minimum(g_iota, nvalid - 1.0)
    te_ref[...] = ge
    xg_ref[...] = xg.astype(jnp.int32)
    valid_ref[...] = validg.astype(jnp.int32)


def _router(flat, router_w, ut_const):
    return pl.pallas_call(
        _router_body,
        in_specs=[
            pl.BlockSpec((T, D), lambda: (0, 0)),
            pl.BlockSpec((D, E), lambda: (0, 0)),
            pl.BlockSpec((T, T), lambda: (0, 0)),
        ],
        out_specs=[
            pl.BlockSpec((2, T), lambda: (0, 0)),
            pl.BlockSpec((2, T), lambda: (0, 0)),
            pl.BlockSpec((1, G), lambda: (0, 0)),
            pl.BlockSpec((1, G), lambda: (0, 0)),
            pl.BlockSpec((1, G), lambda: (0, 0)),
        ],
        out_shape=[
            jax.ShapeDtypeStruct((2, T), jnp.int32),
            jax.ShapeDtypeStruct((2, T), jnp.float32),
            jax.ShapeDtypeStruct((1, G), jnp.int32),
            jax.ShapeDtypeStruct((1, G), jnp.int32),
            jax.ShapeDtypeStruct((1, G), jnp.int32),
        ],
    )(flat, router_w, ut_const)


def _dispatch_sc(flat, pos, w2):
    mesh = plsc.VectorSubcoreMesh(core_axis_name="c", subcore_axis_name="s")

    @functools.partial(
        pl.kernel, mesh=mesh,
        out_type=(
            jax.ShapeDtypeStruct((P, D), jnp.float32),
            jax.ShapeDtypeStruct((P,), jnp.float32),
        ),
        scratch_types=[
            pltpu.VMEM((TPW,), jnp.int32),
            pltpu.VMEM((TPW,), jnp.int32),
            pltpu.VMEM((TPW,), jnp.float32),
            pltpu.VMEM((TPW,), jnp.float32),
            pltpu.VMEM((TPW, D), jnp.float32),
            pltpu.SemaphoreType.DMA,
        ],
    )
    def k(flat_hbm, pos_hbm, w2_hbm, xs_hbm, ws_hbm,
          idx0_v, idx1_v, wv0_v, wv1_v, rows_v, sem):
        wid = lax.axis_index("s") * 2 + lax.axis_index("c")
        tbase = wid * TPW
        pltpu.sync_copy(pos_hbm.at[0, pl.ds(tbase, TPW)], idx0_v)
        pltpu.sync_copy(pos_hbm.at[1, pl.ds(tbase, TPW)], idx1_v)
        pltpu.sync_copy(w2_hbm.at[0, pl.ds(tbase, TPW)], wv0_v)
        pltpu.sync_copy(w2_hbm.at[1, pl.ds(tbase, TPW)], wv1_v)
        pltpu.sync_copy(flat_hbm.at[pl.ds(tbase, TPW)], rows_v)
        c0 = pltpu.async_copy(rows_v, xs_hbm.at[idx0_v], sem)
        c1 = pltpu.async_copy(rows_v, xs_hbm.at[idx1_v], sem)
        c2 = pltpu.async_copy(wv0_v, ws_hbm.at[idx0_v], sem)
        c3 = pltpu.async_copy(wv1_v, ws_hbm.at[idx1_v], sem)
        c0.wait()
        c1.wait()
        c2.wait()
        c3.wait()

    return k(flat, pos, w2)


def _gemm_body(te_ref, xg_ref, valid_ref, xs_ref, wg_ref, wu_ref, wd_ref,
               ws_ref, y_ref, acc_ref):
    g = pl.program_id(0)
    j = pl.program_id(1)

    @pl.when(valid_ref[g] == 1)
    def _():
        x = xs_ref[...]
        gate = lax.dot_general(x, wg_ref[0], (((1,), (0,)), ((), ())),
                               preferred_element_type=jnp.float32)
        up = lax.dot_general(x, wu_ref[0], (((1,), (0,)), ((), ())),
                             preferred_element_type=jnp.float32)
        gated = (gate * lax.logistic(gate)) * up
        contrib = lax.dot_general(gated, wd_ref[0], (((1,), (0,)), ((), ())),
                                  preferred_element_type=jnp.float32)

        @pl.when(j == 0)
        def _():
            acc_ref[...] = jnp.zeros_like(acc_ref)

        acc_ref[...] += contrib

        @pl.when(j == NFF - 1)
        def _():
            y_ref[...] = acc_ref[...] * ws_ref[...]


def _gemm(te, xg, valid, xs, gate_up_proj, down_proj, ws):
    def jc(j, valid_g):
        return jnp.where(valid_g == 1, j, NFF - 1)

    grid_spec = pltpu.PrefetchScalarGridSpec(
        num_scalar_prefetch=3,
        grid=(G, NFF),
        in_specs=[
            pl.BlockSpec((B, D), lambda g, j, te, xg, v: (xg[g], 0)),
            pl.BlockSpec((1, D, F),
                         lambda g, j, te, xg, v: (te[g], 0, jc(j, v[g]))),
            pl.BlockSpec((1, D, F),
                         lambda g, j, te, xg, v: (te[g], 0, jc(j, v[g]) + NFF)),
            pl.BlockSpec((1, F, D),
                         lambda g, j, te, xg, v: (te[g], jc(j, v[g]), 0)),
            pl.BlockSpec((B, 1), lambda g, j, te, xg, v: (xg[g], 0)),
        ],
        out_specs=pl.BlockSpec((B, D), lambda g, j, te, xg, v: (xg[g], 0)),
        scratch_shapes=[pltpu.VMEM((B, D), jnp.float32)],
    )
    return pl.pallas_call(
        _gemm_body,
        grid_spec=grid_spec,
        out_shape=jax.ShapeDtypeStruct((P, D), jnp.float32),
        compiler_params=pltpu.CompilerParams(
            dimension_semantics=("arbitrary", "arbitrary"),
        ),
    )(te, xg, valid, xs, gate_up_proj, gate_up_proj, down_proj,
      ws.reshape(P, 1))


_CH = 32


def _combine_sc(y, pos):
    mesh = plsc.VectorSubcoreMesh(core_axis_name="c", subcore_axis_name="s")

    @functools.partial(
        pl.kernel, mesh=mesh,
        out_type=jax.ShapeDtypeStruct((T, D), jnp.float32),
        scratch_types=[
            pltpu.VMEM((_CH,), jnp.int32),
            pltpu.VMEM((_CH,), jnp.int32),
            pltpu.VMEM((_CH, D), jnp.float32),
            pltpu.VMEM((_CH, D), jnp.float32),
            pltpu.SemaphoreType.DMA,
        ],
    )
    def k(y_hbm, pos_hbm, out_hbm, idx0_v, idx1_v, y0_v, y1_v, sem):
        wid = lax.axis_index("s") * 2 + lax.axis_index("c")
        for ch in range(TPW // _CH):
            tbase = wid * TPW + ch * _CH
            pltpu.sync_copy(pos_hbm.at[0, pl.ds(tbase, _CH)], idx0_v)
            pltpu.sync_copy(pos_hbm.at[1, pl.ds(tbase, _CH)], idx1_v)
            c0 = pltpu.async_copy(y_hbm.at[idx0_v], y0_v, sem)
            c1 = pltpu.async_copy(y_hbm.at[idx1_v], y1_v, sem)
            c0.wait()
            c1.wait()

            def row(r, _):
                for c in range(D // 16):
                    y0_v[r, pl.ds(c * 16, 16)] += y1_v[r, pl.ds(c * 16, 16)]
                return 0

            lax.fori_loop(0, _CH, row, 0)
            pltpu.sync_copy(y0_v, out_hbm.at[pl.ds(tbase, _CH)])

    return k(y, pos)


@jax.jit
def kernel(hidden_states, router_w, gate_up_proj, down_proj):
    b, s, d = hidden_states.shape
    flat = hidden_states.reshape(-1, d)
    ut_const = jnp.asarray(_np.triu(_np.ones((T, T), _np.float32), k=1),
                           dtype=jnp.bfloat16)
    pos, w2, te, xg, valid = _router(flat, router_w, ut_const)
    xs, ws = _dispatch_sc(flat, pos, w2)
    y = _gemm(te.reshape(G), xg.reshape(G), valid.reshape(G),
              xs, gate_up_proj, down_proj, ws)
    out = _combine_sc(y, pos)
    return out.reshape(b, s, d)
